# baseline (speedup 1.0000x reference)
import functools

import jax
import jax.numpy as jnp
from jax import lax
from jax.experimental import pallas as pl
from jax.experimental.pallas import tpu as pltpu
from jax.experimental.pallas import tpu_sc as plsc

VOCAB = 100000
N_WORD = 64
B = 4096
L = 50

_INFO = plsc.get_sparse_core_info()
_NC = _INFO.num_cores
_NS = _INFO.num_subcores
_NW = _NC * _NS
_PER_W = (B // _NW) * L
_BB = 128
_DT = N_WORD // 8


@functools.partial(
    pl.kernel,
    mesh=plsc.VectorSubcoreMesh(core_axis_name="c", subcore_axis_name="s"),
    out_type=jax.ShapeDtypeStruct((L, _DT, _NW, 8, _BB), jnp.float32),
    scratch_types=[
        pltpu.VMEM((_PER_W,), jnp.int32),
        [pltpu.VMEM((_BB,), jnp.int32) for _ in range(2)],
        [pltpu.VMEM((_BB, N_WORD), jnp.float32) for _ in range(2)],
        [pltpu.VMEM((_DT, 8, _BB), jnp.float32) for _ in range(2)],
        [pltpu.SemaphoreType.DMA for _ in range(2)],
        [pltpu.SemaphoreType.DMA for _ in range(2)],
    ],
    compiler_params=pltpu.CompilerParams(
        use_tc_tiling_on_sc=False, needs_layout_passes=False),
)
def _gather_kernel(idx_hbm, table_hbm, out_hbm, idx_v, tidx, rows, tiles,
                   gsem, ssem):
    wid = lax.axis_index("s") * _NC + lax.axis_index("c")
    pltpu.sync_copy(idx_hbm.at[pl.ds(wid * _PER_W, _PER_W)], idx_v)
    lanes = lax.iota(jnp.int32, 16)
    lanes50 = lanes * L

    def build_tidx(b, l):
        for j in range(8):
            sel = plsc.load_gather(idx_v, [lanes50 + (j * 16 * L + l)])
            tidx[b][pl.ds(j * 16, 16)] = sel

    def fire_gather(b):
        return pltpu.async_copy(table_hbm.at[tidx[b]], rows[b], gsem[b])

    def transpose(b):
        rb = rows[b]
        tb = tiles[b]
        for dt in range(8):
            for dr in range(8):
                cols = jnp.full((16,), dt * 8 + dr, jnp.int32)
                for j in range(8):
                    v = plsc.load_gather(rb, [lanes + j * 16, cols])
                    tb[dt, dr, pl.ds(j * 16, 16)] = v

    def fire_store(b, l):
        return pltpu.async_copy(tiles[b], out_hbm.at[l, :, wid], ssem[b])

    def wait_gather(b):
        pltpu.make_async_copy(table_hbm.at[tidx[b]], rows[b], gsem[b]).wait()

    def wait_store(b, l):
        pltpu.make_async_copy(tiles[b], out_hbm.at[l, :, wid], ssem[b]).wait()

    build_tidx(0, 0)
    fire_gather(0)

    def step(i, _):
        for par in range(2):
            l = i * 2 + par
            nl = l + 1

            @pl.when(nl < L)
            def _():
                build_tidx(1 - par, nl)
                fire_gather(1 - par)

            wait_gather(par)

            @pl.when(l >= 2)
            def _():
                wait_store(par, l)

            transpose(par)
            fire_store(par, l)
        return 0

    lax.fori_loop(0, L // 2, step, 0)
    wait_store(0, L - 2)
    wait_store(1, L - 1)


def kernel(val_tok, embedding_weight):
    flat_idx = val_tok.reshape(B * L).astype(jnp.int32)
    z = _gather_kernel(flat_idx, embedding_weight)
    return z.transpose(2, 4, 0, 1, 3).reshape(B, L, N_WORD)


# speedup vs baseline: 1.9105x; 1.9105x over previous
import functools

import jax
import jax.numpy as jnp
from jax import lax
from jax.experimental import pallas as pl
from jax.experimental.pallas import tpu as pltpu
from jax.experimental.pallas import tpu_sc as plsc

VOCAB = 100000
N_WORD = 64
B = 4096
L = 50

_INFO = plsc.get_sparse_core_info()
_NC = _INFO.num_cores
_NS = _INFO.num_subcores
_NW = _NC * _NS
_PER_W = (B // _NW) * L
_BB = 128
_DT = N_WORD // 8


@functools.partial(
    pl.kernel,
    mesh=plsc.VectorSubcoreMesh(core_axis_name="c", subcore_axis_name="s"),
    out_type=jax.ShapeDtypeStruct((L, _DT, _NW, 8, _BB), jnp.float32),
    scratch_types=[
        pltpu.VMEM((_PER_W,), jnp.int32),
        [pltpu.VMEM((_BB,), jnp.int32) for _ in range(2)],
        [pltpu.VMEM((_BB, N_WORD), jnp.float32) for _ in range(2)],
        [pltpu.VMEM((N_WORD, _BB + 1), jnp.float32) for _ in range(2)],
        [pltpu.SemaphoreType.DMA for _ in range(2)],
        [pltpu.SemaphoreType.DMA for _ in range(2)],
    ],
    compiler_params=pltpu.CompilerParams(
        use_tc_tiling_on_sc=False, needs_layout_passes=False),
)
def _gather_kernel(idx_hbm, table_hbm, out_hbm, idx_v, tidx, rows, tiles,
                   gsem, ssem):
    wid = lax.axis_index("s") * _NC + lax.axis_index("c")
    pltpu.sync_copy(idx_hbm.at[pl.ds(wid * _PER_W, _PER_W)], idx_v)
    lanes = lax.iota(jnp.int32, 16)
    lanes50 = lanes * L

    def build_tidx(b, l):
        for j in range(8):
            sel = plsc.load_gather(idx_v, [lanes50 + (j * 16 * L + l)])
            tidx[b][pl.ds(j * 16, 16)] = sel

    def fire_gather(b):
        return pltpu.async_copy(table_hbm.at[tidx[b]], rows[b], gsem[b])

    def transpose(b):
        rb = rows[b]
        tb = tiles[b]
        dvecs = [lanes + 16 * q for q in range(4)]
        for bc in range(_BB):
            bcvec = jnp.full((16,), bc, jnp.int32)
            for q in range(4):
                v = rb[bc, pl.ds(q * 16, 16)]
                plsc.store_scatter(tb, [dvecs[q], bcvec], v)

    def fire_store(b, l):
        for dt in range(_DT):
            pltpu.async_copy(
                tiles[b].at[pl.ds(dt * 8, 8), pl.ds(0, _BB)],
                out_hbm.at[l, dt, wid], ssem[b])

    def wait_gather(b):
        pltpu.make_async_copy(table_hbm.at[tidx[b]], rows[b], gsem[b]).wait()

    def wait_store(b, l):
        for dt in range(_DT):
            pltpu.make_async_copy(
                tiles[b].at[pl.ds(dt * 8, 8), pl.ds(0, _BB)],
                out_hbm.at[l, dt, wid], ssem[b]).wait()

    build_tidx(0, 0)
    fire_gather(0)

    def step(i, _):
        for par in range(2):
            l = i * 2 + par
            nl = l + 1

            @pl.when(nl < L)
            def _():
                build_tidx(1 - par, nl)
                fire_gather(1 - par)

            wait_gather(par)

            @pl.when(l >= 2)
            def _():
                wait_store(par, l)

            transpose(par)
            fire_store(par, l)
        return 0

    lax.fori_loop(0, L // 2, step, 0)
    wait_store(0, L - 2)
    wait_store(1, L - 1)


def kernel(val_tok, embedding_weight):
    flat_idx = val_tok.reshape(B * L).astype(jnp.int32)
    z = _gather_kernel(flat_idx, embedding_weight)
    return z.transpose(2, 4, 0, 1, 3).reshape(B, L, N_WORD)


# speedup vs baseline: 1.9423x; 1.0166x over previous
import functools

import jax
import jax.numpy as jnp
from jax import lax
from jax.experimental import pallas as pl
from jax.experimental.pallas import tpu as pltpu
from jax.experimental.pallas import tpu_sc as plsc

VOCAB = 100000
N_WORD = 64
B = 4096
L = 50

_INFO = plsc.get_sparse_core_info()
_NC = _INFO.num_cores
_NS = _INFO.num_subcores
_NW = _NC * _NS
_PER_W = (B // _NW) * L
_BB = 128
_DT = N_WORD // 8


@functools.partial(
    pl.kernel,
    mesh=plsc.VectorSubcoreMesh(core_axis_name="c", subcore_axis_name="s"),
    out_type=jax.ShapeDtypeStruct((L, _DT, _NW, 8, _BB), jnp.float32),
    scratch_types=[
        pltpu.VMEM((_PER_W,), jnp.int32),
        [pltpu.VMEM((_BB,), jnp.int32) for _ in range(2)],
        [pltpu.VMEM((_BB, N_WORD), jnp.float32) for _ in range(2)],
        [pltpu.VMEM((_DT, 8, _BB + 1), jnp.float32) for _ in range(2)],
        [pltpu.SemaphoreType.DMA for _ in range(2)],
        [pltpu.SemaphoreType.DMA for _ in range(2)],
    ],
    compiler_params=pltpu.CompilerParams(
        use_tc_tiling_on_sc=False, needs_layout_passes=False),
)
def _gather_kernel(idx_hbm, table_hbm, out_hbm, idx_v, tidx, rows, tiles,
                   gsem, ssem):
    wid = lax.axis_index("s") * _NC + lax.axis_index("c")
    pltpu.sync_copy(idx_hbm.at[pl.ds(wid * _PER_W, _PER_W)], idx_v)
    lanes = lax.iota(jnp.int32, 16)
    lanes50 = lanes * L

    def build_tidx(b, l):
        for j in range(8):
            sel = plsc.load_gather(idx_v, [lanes50 + (j * 16 * L + l)])
            tidx[b][pl.ds(j * 16, 16)] = sel

    def fire_gather(b):
        return pltpu.async_copy(table_hbm.at[tidx[b]], rows[b], gsem[b])

    dvecs = [lanes // 8 + 2 * q for q in range(4)]
    drvec = lanes % 8

    def transpose(b):
        rb = rows[b]
        tb = tiles[b]
        for g in range(0, _BB, 4):
            vs = []
            for bc in range(g, g + 4):
                for q in range(4):
                    vs.append((rb[bc, pl.ds(q * 16, 16)], q,
                               jnp.full((16,), bc, jnp.int32)))
            for v, q, bcvec in vs:
                plsc.store_scatter(tb, [dvecs[q], drvec, bcvec], v)

    def fire_store(b, l):
        return pltpu.async_copy(
            tiles[b].at[:, :, pl.ds(0, _BB)], out_hbm.at[l, :, wid], ssem[b])

    def wait_gather(b):
        pltpu.make_async_copy(table_hbm.at[tidx[b]], rows[b], gsem[b]).wait()

    def wait_store(b, l):
        pltpu.make_async_copy(
            tiles[b].at[:, :, pl.ds(0, _BB)], out_hbm.at[l, :, wid],
            ssem[b]).wait()

    build_tidx(0, 0)
    fire_gather(0)

    def step(i, _):
        for par in range(2):
            l = i * 2 + par
            nl = l + 1

            @pl.when(nl < L)
            def _():
                build_tidx(1 - par, nl)
                fire_gather(1 - par)

            wait_gather(par)

            @pl.when(l >= 2)
            def _():
                wait_store(par, l)

            transpose(par)
            fire_store(par, l)
        return 0

    lax.fori_loop(0, L // 2, step, 0)
    wait_store(0, L - 2)
    wait_store(1, L - 1)


def kernel(val_tok, embedding_weight):
    flat_idx = val_tok.reshape(B * L).astype(jnp.int32)
    z = _gather_kernel(flat_idx, embedding_weight)
    return z.transpose(2, 4, 0, 1, 3).reshape(B, L, N_WORD)


# speedup vs baseline: 3.1947x; 1.6448x over previous
import functools

import jax
import jax.numpy as jnp
from jax import lax
from jax.experimental import pallas as pl
from jax.experimental.pallas import tpu as pltpu
from jax.experimental.pallas import tpu_sc as plsc

VOCAB = 100000
N_WORD = 64
B = 4096
L = 50

_INFO = plsc.get_sparse_core_info()
_NC = _INFO.num_cores
_NS = _INFO.num_subcores
_NW = _NC * _NS
_PER_W = (B // _NW) * L
_BB = 128
_DT = N_WORD // 8


@functools.partial(
    pl.kernel,
    mesh=plsc.VectorSubcoreMesh(core_axis_name="c", subcore_axis_name="s"),
    out_type=jax.ShapeDtypeStruct((L, _DT, _NW, 8, _BB), jnp.float32),
    scratch_types=[
        pltpu.VMEM((_PER_W,), jnp.int32),
        [pltpu.VMEM((_BB,), jnp.int32) for _ in range(2)],
        [pltpu.VMEM((_BB, N_WORD), jnp.float32) for _ in range(2)],
        [pltpu.VMEM((_DT, 8, _BB + 1), jnp.float32) for _ in range(2)],
        [pltpu.SemaphoreType.DMA for _ in range(2)],
        [pltpu.SemaphoreType.DMA for _ in range(2)],
    ],
    compiler_params=pltpu.CompilerParams(
        use_tc_tiling_on_sc=False, needs_layout_passes=False),
)
def _gather_kernel(idx_hbm, table_hbm, out_hbm, idx_v, tidx, rows, tiles,
                   gsem, ssem):
    wid = lax.axis_index("s") * _NC + lax.axis_index("c")
    pltpu.sync_copy(idx_hbm.at[pl.ds(wid * _PER_W, _PER_W)], idx_v)
    lanes = lax.iota(jnp.int32, 16)
    lanes50 = lanes * L

    def build_tidx(b, l):
        for j in range(8):
            sel = plsc.load_gather(idx_v, [lanes50 + (j * 16 * L + l)])
            tidx[b][pl.ds(j * 16, 16)] = sel

    def fire_gather(b):
        return pltpu.async_copy(table_hbm.at[tidx[b]], rows[b], gsem[b])

    dvecs = [lanes // 8 + 2 * q for q in range(4)]
    drvec = lanes % 8

    def transpose(b):
        rb = rows[b]
        tb = tiles[b]

        @plsc.parallel_loop(0, _BB, 1, unroll=8)
        def _(bc):
            bcvec = jnp.zeros((16,), jnp.int32) + bc
            for q in range(4):
                v = rb[bc, pl.ds(q * 16, 16)]
                plsc.store_scatter(tb, [dvecs[q], drvec, bcvec], v)

    def fire_store(b, l):
        return pltpu.async_copy(
            tiles[b].at[:, :, pl.ds(0, _BB)], out_hbm.at[l, :, wid], ssem[b])

    def wait_gather(b):
        pltpu.make_async_copy(table_hbm.at[tidx[b]], rows[b], gsem[b]).wait()

    def wait_store(b, l):
        pltpu.make_async_copy(
            tiles[b].at[:, :, pl.ds(0, _BB)], out_hbm.at[l, :, wid],
            ssem[b]).wait()

    build_tidx(0, 0)
    fire_gather(0)

    def step(i, _):
        for par in range(2):
            l = i * 2 + par
            nl = l + 1

            @pl.when(nl < L)
            def _():
                build_tidx(1 - par, nl)
                fire_gather(1 - par)

            wait_gather(par)

            @pl.when(l >= 2)
            def _():
                wait_store(par, l)

            transpose(par)
            fire_store(par, l)
        return 0

    lax.fori_loop(0, L // 2, step, 0)
    wait_store(0, L - 2)
    wait_store(1, L - 1)


def kernel(val_tok, embedding_weight):
    flat_idx = val_tok.reshape(B * L).astype(jnp.int32)
    z = _gather_kernel(flat_idx, embedding_weight)
    return z.transpose(2, 4, 0, 1, 3).reshape(B, L, N_WORD)


# speedup vs baseline: 3.4225x; 1.0713x over previous
import functools

import jax
import jax.numpy as jnp
from jax import lax
from jax.experimental import pallas as pl
from jax.experimental.pallas import tpu as pltpu
from jax.experimental.pallas import tpu_sc as plsc

VOCAB = 100000
N_WORD = 64
B = 4096
L = 50

_INFO = plsc.get_sparse_core_info()
_NC = _INFO.num_cores
_NS = _INFO.num_subcores
_NW = _NC * _NS
_PER_W = (B // _NW) * L
_BB = 128
_DT = N_WORD // 8


@functools.partial(
    pl.kernel,
    mesh=plsc.VectorSubcoreMesh(core_axis_name="c", subcore_axis_name="s"),
    out_type=jax.ShapeDtypeStruct((L, _DT, _NW, 8, _BB), jnp.float32),
    scratch_types=[
        pltpu.VMEM((_PER_W,), jnp.int32),
        [pltpu.VMEM((_BB,), jnp.int32) for _ in range(4)],
        [pltpu.VMEM((_BB, N_WORD), jnp.float32) for _ in range(4)],
        [pltpu.VMEM((_DT, 8, _BB + 1), jnp.float32) for _ in range(2)],
        [pltpu.SemaphoreType.DMA for _ in range(4)],
        [pltpu.SemaphoreType.DMA for _ in range(2)],
    ],
    compiler_params=pltpu.CompilerParams(
        use_tc_tiling_on_sc=False, needs_layout_passes=False),
)
def _gather_kernel(idx_hbm, table_hbm, out_hbm, idx_v, tidx, rows, tiles,
                   gsem, ssem):
    wid = lax.axis_index("s") * _NC + lax.axis_index("c")
    pltpu.sync_copy(idx_hbm.at[pl.ds(wid * _PER_W, _PER_W)], idx_v)
    lanes = lax.iota(jnp.int32, 16)
    lanes50 = lanes * L

    def build_tidx(b, l):
        for j in range(8):
            sel = plsc.load_gather(idx_v, [lanes50 + (j * 16 * L + l)])
            tidx[b][pl.ds(j * 16, 16)] = sel

    def fire_gather(b):
        return pltpu.async_copy(table_hbm.at[tidx[b]], rows[b], gsem[b])

    dvecs = [lanes // 8 + 2 * q for q in range(4)]
    drvec = lanes % 8

    def transpose(bg, bt):
        rb = rows[bg]
        tb = tiles[bt]

        @plsc.parallel_loop(0, _BB, 1, unroll=8)
        def _(bc):
            bcvec = jnp.zeros((16,), jnp.int32) + bc
            for q in range(4):
                v = rb[bc, pl.ds(q * 16, 16)]
                plsc.store_scatter(tb, [dvecs[q], drvec, bcvec], v)

    def fire_store(b, l):
        return pltpu.async_copy(
            tiles[b].at[:, :, pl.ds(0, _BB)], out_hbm.at[l, :, wid], ssem[b])

    def wait_gather(b):
        pltpu.make_async_copy(table_hbm.at[tidx[b]], rows[b], gsem[b]).wait()

    def wait_store(b, l):
        pltpu.make_async_copy(
            tiles[b].at[:, :, pl.ds(0, _BB)], out_hbm.at[l, :, wid],
            ssem[b]).wait()

    def substep(l, g4, t2, fire):
        if fire:
            build_tidx((g4 + 2) % 4, l + 2)
            fire_gather((g4 + 2) % 4)
        wait_gather(g4)

        @pl.when(l >= 2)
        def _():
            wait_store(t2, l)

        transpose(g4, t2)
        fire_store(t2, l)

    build_tidx(0, 0)
    fire_gather(0)
    build_tidx(1, 1)
    fire_gather(1)

    def step(i, _):
        for j in range(4):
            substep(i * 4 + j, j, j % 2, True)
        return 0

    lax.fori_loop(0, (L - 2) // 4, step, 0)
    substep(L - 2, (L - 2) % 4, 0, False)
    substep(L - 1, (L - 1) % 4, 1, False)
    wait_store(0, L - 2)
    wait_store(1, L - 1)


def kernel(val_tok, embedding_weight):
    flat_idx = val_tok.reshape(B * L).astype(jnp.int32)
    z = _gather_kernel(flat_idx, embedding_weight)
    return z.transpose(2, 4, 0, 1, 3).reshape(B, L, N_WORD)
